# baseline (speedup 1.0000x reference)
import functools

import jax
import jax.numpy as jnp
from jax import lax
from jax.experimental import pallas as pl
from jax.experimental.pallas import tpu as pltpu
from jax.experimental.pallas import tpu_sc as plsc

B = 16384
L = 20
D_USER, D_ITEM, D_HOUR, D_TEXT, D_PRICE = 20, 20, 5, 17, 10
D_OUT = 73
NC, NS, LANES = 2, 16, 16
NW = NC * NS
RPT = B // NW
HROW = 128
SENT = -1

_mesh = plsc.VectorSubcoreMesh(
    core_axis_name="c", subcore_axis_name="s", num_cores=NC, num_subcores=NS
)


def _iota16():
  return lax.iota(jnp.int32, LANES)


@functools.partial(
    pl.kernel,
    out_type=jax.ShapeDtypeStruct((B, D_OUT), jnp.float32),
    mesh=_mesh,
    compiler_params=pltpu.CompilerParams(
        needs_layout_passes=False, use_tc_tiling_on_sc=False,
        disable_bounds_checks=True),
    scratch_types=dict(
        idx_u=pltpu.VMEM((RPT,), jnp.int32),
        idx_i=pltpu.VMEM((RPT,), jnp.int32),
        idx_h=pltpu.VMEM((RPT,), jnp.int32),
        idx_p=pltpu.VMEM((RPT,), jnp.int32),
        it_a=pltpu.VMEM((RPT,), jnp.int32),
        it_b=pltpu.VMEM((RPT,), jnp.int32),
        is_a=pltpu.VMEM((RPT,), jnp.int32),
        is_b=pltpu.VMEM((RPT,), jnp.int32),
        tt=pltpu.VMEM((RPT * L,), jnp.int32),
        ctr_v=pltpu.VMEM((RPT,), jnp.float32),
        rows_u=pltpu.VMEM((RPT, 32), jnp.float32),
        rows_i=pltpu.VMEM((RPT, 32), jnp.float32),
        rows_p=pltpu.VMEM((RPT, 16), jnp.float32),
        tab_h=pltpu.VMEM((D_HOUR * (24 + 1),), jnp.float32),
        recip=pltpu.VMEM((RPT,), jnp.float32),
        out_v=pltpu.VMEM((RPT, D_OUT), jnp.float32),
        acc_s=pltpu.VMEM_SHARED((NS * RPT, 32), jnp.float32),
        sem_a=pltpu.SemaphoreType.DMA,
        sem_b=pltpu.SemaphoreType.DMA,
        sem_p=pltpu.SemaphoreType.DMA,
    ),
)
def _emb_kernel(W_user, W_item, W_hour, W_text, W_price, norm_ctr,
                cat_user_id, cat_item_id, int_hour, text_flat, disc_price,
                out_hbm, *, idx_u, idx_i, idx_h, idx_p,
                it_a, it_b, is_a, is_b, tt, ctr_v, rows_u, rows_i, rows_p,
                tab_h, recip, out_v, acc_s, sem_a, sem_b, sem_p):
  c = lax.axis_index("c")
  s = lax.axis_index("s")
  wid = c * NS + s
  base = wid * RPT
  sbase = s * RPT

  w_user, w_item, w_text, w_price = W_user, W_item, W_text, W_price

  pltpu.sync_copy(cat_user_id.at[pl.ds(base, RPT)], idx_u)
  pltpu.sync_copy(cat_item_id.at[pl.ds(base, RPT)], idx_i)
  pltpu.sync_copy(int_hour.at[pl.ds(base, RPT)], idx_h)
  pltpu.sync_copy(disc_price.at[pl.ds(base, RPT)], idx_p)
  pltpu.sync_copy(norm_ctr.at[pl.ds(base, RPT)], ctr_v)
  pltpu.sync_copy(text_flat.at[pl.ds(base * L, RPT * L)], tt)
  pltpu.sync_copy(W_hour, tab_h)

  iota = _iota16()

  def _zero_body(k, _):
    z = jnp.zeros((LANES,), jnp.float32)
    rows_u[k, pl.ds(0, LANES)] = z
    rows_u[k, pl.ds(LANES, LANES)] = z
    return 0
  lax.fori_loop(0, RPT, _zero_body, 0)
  pltpu.sync_copy(rows_u, acc_s.at[pl.ds(sbase, RPT)])

  cp_u = pltpu.async_copy(w_user.at[idx_u], rows_u, sem_a)
  cp_i = pltpu.async_copy(w_item.at[idx_i], rows_i, sem_b)
  cp_p = pltpu.async_copy(w_price.at[idx_p], rows_p, sem_p)

  def _cnt_body(k, _):
    rr = (k * LANES + iota) * L
    cnt = jnp.zeros((LANES,), jnp.float32)
    for t in range(L):
      tok = plsc.load_gather(tt, [rr + t])
      cnt += jnp.where(tok != 0, 1.0, 0.0).astype(jnp.float32)
    recip[pl.ds(k * LANES, LANES)] = 1.0 / jnp.maximum(cnt, 1.0)
    return 0
  lax.fori_loop(0, RPT // LANES, _cnt_body, 0)

  def _copy_feat(src, d, col0):
    def body(i, _):
      lin = iota + i * LANES
      r = lax.div(lin, d)
      cc = lin - r * d
      v = plsc.load_gather(src, [r, cc])
      plsc.store_scatter(out_v, [r, col0 + cc], v)
      return 0
    lax.fori_loop(0, RPT * d // LANES, body, 0)

  cp_u.wait()
  _copy_feat(rows_u, D_USER, 0)
  cp_i.wait()
  _copy_feat(rows_i, D_ITEM, D_USER)
  cp_p.wait()
  _copy_feat(rows_p, D_PRICE, 62)

  def _hour_body(i, _):
    lin = iota + i * LANES
    r = lax.div(lin, D_HOUR)
    cc = lin - r * D_HOUR
    hh = plsc.load_gather(idx_h, [r])
    v = plsc.load_gather(tab_h, [hh * D_HOUR + cc])
    plsc.store_scatter(out_v, [r, 40 + cc], v)
    return 0
  lax.fori_loop(0, RPT * D_HOUR // LANES, _hour_body, 0)

  def _ctr_body(i, _):
    r = iota + i * LANES
    v = ctr_v[pl.ds(i * LANES, LANES)]
    plsc.store_scatter(out_v, [r, jnp.full((LANES,), D_OUT - 1, jnp.int32)], v)
    return 0
  lax.fori_loop(0, RPT // LANES, _ctr_body, 0)

  tbufs = (rows_u, rows_i)
  tsems = (sem_a, sem_b)
  tidx = (it_a, it_b)
  tsct = (is_a, is_b)

  def _gen_idx(t):
    def body(k, _):
      rloc = k * LANES + iota
      tok = plsc.load_gather(tt, [rloc * L + t])
      sl = pl.ds(k * LANES, LANES)
      tidx[t % 2][sl] = tok
      tsct[t % 2][sl] = jnp.where(tok == 0, jnp.int32(SENT), sbase + rloc)
      return 0
    lax.fori_loop(0, RPT // LANES, body, 0)

  _gen_idx(0)
  pend = [pltpu.async_copy(w_text.at[tidx[0]], tbufs[0], tsems[0])]
  _gen_idx(1)
  pend.append(pltpu.async_copy(w_text.at[tidx[1]], tbufs[1], tsems[1]))

  for t in range(L):
    pend.pop(0).wait()
    pltpu.sync_copy(
        tbufs[t % 2],
        acc_s.at[plsc.Indices(tsct[t % 2], ignored_value=SENT)],
        add=True)
    if t + 2 < L:
      _gen_idx(t + 2)
      pend.append(
          pltpu.async_copy(w_text.at[tidx[t % 2]], tbufs[t % 2], tsems[t % 2]))

  acc_v = rows_u
  pltpu.sync_copy(acc_s.at[pl.ds(sbase, RPT)], acc_v)

  def _text_body(i, _):
    lin = iota + i * LANES
    r = lax.div(lin, D_TEXT)
    cc = lin - r * D_TEXT
    a = plsc.load_gather(acc_v, [r, cc])
    rc = plsc.load_gather(recip, [r])
    plsc.store_scatter(out_v, [r, 45 + cc], a * rc)
    return 0
  lax.fori_loop(0, RPT * D_TEXT // LANES, _text_body, 0)

  pltpu.sync_copy(out_v, out_hbm.at[pl.ds(base, RPT)])


_PAD_BS = 8192


def _widen_body(u_ref, i_ref, uo_ref, io_ref):
  uo_ref[:, 0:D_USER] = u_ref[...]
  io_ref[:, 0:D_ITEM] = i_ref[...]


def _widen_tables(W_user, W_item):
  n = W_user.shape[0]
  grid = (n + _PAD_BS - 1) // _PAD_BS
  return pl.pallas_call(
      _widen_body,
      grid=(grid,),
      in_specs=[
          pl.BlockSpec((_PAD_BS, D_USER), lambda i: (i, 0)),
          pl.BlockSpec((_PAD_BS, D_ITEM), lambda i: (i, 0)),
      ],
      out_specs=[
          pl.BlockSpec((_PAD_BS, 32), lambda i: (i, 0)),
          pl.BlockSpec((_PAD_BS, 32), lambda i: (i, 0)),
      ],
      out_shape=[
          jax.ShapeDtypeStruct((n, 32), jnp.float32),
          jax.ShapeDtypeStruct((n, 32), jnp.float32),
      ],
  )(W_user, W_item)


def kernel(W_user, W_item, W_hour, W_text, W_price, norm_ctr, cat_user_id,
           cat_item_id, int_hour, text_title, disc_clip_price):
  w_user, w_item = _widen_tables(W_user, W_item)
  w_text = jnp.pad(W_text, ((0, 0), (0, 32 - D_TEXT)))
  w_price = jnp.pad(W_price, ((0, 0), (0, 16 - D_PRICE)))
  return _emb_kernel(
      w_user, w_item, jnp.reshape(W_hour, (-1,)), w_text, w_price,
      norm_ctr, cat_user_id, cat_item_id, int_hour,
      jnp.reshape(text_title, (-1,)), disc_clip_price)


# speedup vs baseline: 1.0240x; 1.0240x over previous
import functools

import jax
import jax.numpy as jnp
from jax import lax
from jax.experimental import pallas as pl
from jax.experimental.pallas import tpu as pltpu
from jax.experimental.pallas import tpu_sc as plsc

B = 16384
L = 20
D_USER, D_ITEM, D_HOUR, D_TEXT, D_PRICE = 20, 20, 5, 17, 10
D_OUT = 73
NC, NS, LANES = 2, 16, 16
NW = NC * NS
RPT = B // NW
HROW = 128
SENT = -1
DW = 24

_mesh = plsc.VectorSubcoreMesh(
    core_axis_name="c", subcore_axis_name="s", num_cores=NC, num_subcores=NS
)


def _iota16():
  return lax.iota(jnp.int32, LANES)


@functools.partial(
    pl.kernel,
    out_type=jax.ShapeDtypeStruct((B, D_OUT), jnp.float32),
    mesh=_mesh,
    compiler_params=pltpu.CompilerParams(
        needs_layout_passes=False, use_tc_tiling_on_sc=False,
        disable_bounds_checks=True),
    scratch_types=dict(
        idx_u=pltpu.VMEM((RPT,), jnp.int32),
        idx_i=pltpu.VMEM((RPT,), jnp.int32),
        idx_h=pltpu.VMEM((RPT,), jnp.int32),
        idx_p=pltpu.VMEM((RPT,), jnp.int32),
        it_a=pltpu.VMEM((RPT,), jnp.int32),
        it_b=pltpu.VMEM((RPT,), jnp.int32),
        is_a=pltpu.VMEM((RPT,), jnp.int32),
        is_b=pltpu.VMEM((RPT,), jnp.int32),
        tt=pltpu.VMEM((RPT * L,), jnp.int32),
        ctr_v=pltpu.VMEM((RPT,), jnp.float32),
        rows_u=pltpu.VMEM((RPT, DW), jnp.float32),
        rows_i=pltpu.VMEM((RPT, DW), jnp.float32),
        rows_p=pltpu.VMEM((RPT, 16), jnp.float32),
        tab_h=pltpu.VMEM((D_HOUR * (24 + 1),), jnp.float32),
        recip=pltpu.VMEM((RPT,), jnp.float32),
        out_v=pltpu.VMEM((RPT, D_OUT), jnp.float32),
        acc_s=pltpu.VMEM_SHARED((NS * RPT, DW), jnp.float32),
        sem_a=pltpu.SemaphoreType.DMA,
        sem_b=pltpu.SemaphoreType.DMA,
        sem_p=pltpu.SemaphoreType.DMA,
    ),
)
def _emb_kernel(W_user, W_item, W_hour, W_text, W_price, norm_ctr,
                cat_user_id, cat_item_id, int_hour, text_flat, disc_price,
                out_hbm, *, idx_u, idx_i, idx_h, idx_p,
                it_a, it_b, is_a, is_b, tt, ctr_v, rows_u, rows_i, rows_p,
                tab_h, recip, out_v, acc_s, sem_a, sem_b, sem_p):
  c = lax.axis_index("c")
  s = lax.axis_index("s")
  wid = c * NS + s
  base = wid * RPT
  sbase = s * RPT

  w_user, w_item, w_text, w_price = W_user, W_item, W_text, W_price

  pltpu.sync_copy(cat_user_id.at[pl.ds(base, RPT)], idx_u)
  pltpu.sync_copy(cat_item_id.at[pl.ds(base, RPT)], idx_i)
  pltpu.sync_copy(int_hour.at[pl.ds(base, RPT)], idx_h)
  pltpu.sync_copy(disc_price.at[pl.ds(base, RPT)], idx_p)
  pltpu.sync_copy(norm_ctr.at[pl.ds(base, RPT)], ctr_v)
  pltpu.sync_copy(text_flat.at[pl.ds(base * L, RPT * L)], tt)
  pltpu.sync_copy(W_hour, tab_h)

  iota = _iota16()

  def _zero_body(k, _):
    z = jnp.zeros((LANES,), jnp.float32)
    rows_u[k, pl.ds(0, LANES)] = z
    rows_u[k, pl.ds(DW - LANES, LANES)] = z
    return 0
  lax.fori_loop(0, RPT, _zero_body, 0)
  pltpu.sync_copy(rows_u, acc_s.at[pl.ds(sbase, RPT)])

  cp_u = pltpu.async_copy(w_user.at[idx_u], rows_u, sem_a)
  cp_i = pltpu.async_copy(w_item.at[idx_i], rows_i, sem_b)
  cp_p = pltpu.async_copy(w_price.at[idx_p], rows_p, sem_p)

  def _cnt_body(k, _):
    rr = (k * LANES + iota) * L
    cnt = jnp.zeros((LANES,), jnp.float32)
    for t in range(L):
      tok = plsc.load_gather(tt, [rr + t])
      cnt += jnp.where(tok != 0, 1.0, 0.0).astype(jnp.float32)
    recip[pl.ds(k * LANES, LANES)] = 1.0 / jnp.maximum(cnt, 1.0)
    return 0
  lax.fori_loop(0, RPT // LANES, _cnt_body, 0)

  def _copy_feat(src, d, col0):
    def body(i, _):
      lin = iota + i * LANES
      r = lax.div(lin, d)
      cc = lin - r * d
      v = plsc.load_gather(src, [r, cc])
      plsc.store_scatter(out_v, [r, col0 + cc], v)
      return 0
    lax.fori_loop(0, RPT * d // LANES, body, 0)

  cp_u.wait()
  _copy_feat(rows_u, D_USER, 0)
  cp_i.wait()
  _copy_feat(rows_i, D_ITEM, D_USER)
  cp_p.wait()
  _copy_feat(rows_p, D_PRICE, 62)

  def _hour_body(i, _):
    lin = iota + i * LANES
    r = lax.div(lin, D_HOUR)
    cc = lin - r * D_HOUR
    hh = plsc.load_gather(idx_h, [r])
    v = plsc.load_gather(tab_h, [hh * D_HOUR + cc])
    plsc.store_scatter(out_v, [r, 40 + cc], v)
    return 0
  lax.fori_loop(0, RPT * D_HOUR // LANES, _hour_body, 0)

  def _ctr_body(i, _):
    r = iota + i * LANES
    v = ctr_v[pl.ds(i * LANES, LANES)]
    plsc.store_scatter(out_v, [r, jnp.full((LANES,), D_OUT - 1, jnp.int32)], v)
    return 0
  lax.fori_loop(0, RPT // LANES, _ctr_body, 0)

  tbufs = (rows_u, rows_i)
  tsems = (sem_a, sem_b)
  tidx = (it_a, it_b)
  tsct = (is_a, is_b)

  def _gen_idx(t):
    def body(k, _):
      rloc = k * LANES + iota
      tok = plsc.load_gather(tt, [rloc * L + t])
      sl = pl.ds(k * LANES, LANES)
      tidx[t % 2][sl] = tok
      tsct[t % 2][sl] = jnp.where(tok == 0, jnp.int32(SENT), sbase + rloc)
      return 0
    lax.fori_loop(0, RPT // LANES, body, 0)

  _gen_idx(0)
  pend = [pltpu.async_copy(w_text.at[tidx[0]], tbufs[0], tsems[0])]
  _gen_idx(1)
  pend.append(pltpu.async_copy(w_text.at[tidx[1]], tbufs[1], tsems[1]))

  for t in range(L):
    pend.pop(0).wait()
    pltpu.sync_copy(
        tbufs[t % 2],
        acc_s.at[plsc.Indices(tsct[t % 2], ignored_value=SENT)],
        add=True)
    if t + 2 < L:
      _gen_idx(t + 2)
      pend.append(
          pltpu.async_copy(w_text.at[tidx[t % 2]], tbufs[t % 2], tsems[t % 2]))

  acc_v = rows_u
  pltpu.sync_copy(acc_s.at[pl.ds(sbase, RPT)], acc_v)

  def _text_body(i, _):
    lin = iota + i * LANES
    r = lax.div(lin, D_TEXT)
    cc = lin - r * D_TEXT
    a = plsc.load_gather(acc_v, [r, cc])
    rc = plsc.load_gather(recip, [r])
    plsc.store_scatter(out_v, [r, 45 + cc], a * rc)
    return 0
  lax.fori_loop(0, RPT * D_TEXT // LANES, _text_body, 0)

  pltpu.sync_copy(out_v, out_hbm.at[pl.ds(base, RPT)])


def kernel(W_user, W_item, W_hour, W_text, W_price, norm_ctr, cat_user_id,
           cat_item_id, int_hour, text_title, disc_clip_price):
  w_user = jnp.pad(W_user, ((0, 0), (0, DW - D_USER)))
  w_item = jnp.pad(W_item, ((0, 0), (0, DW - D_ITEM)))
  w_text = jnp.pad(W_text, ((0, 0), (0, DW - D_TEXT)))
  w_price = jnp.pad(W_price, ((0, 0), (0, 16 - D_PRICE)))
  return _emb_kernel(
      w_user, w_item, jnp.reshape(W_hour, (-1,)), w_text, w_price,
      norm_ctr, cat_user_id, cat_item_id, int_hour,
      jnp.reshape(text_title, (-1,)), disc_clip_price)


# speedup vs baseline: 1.0930x; 1.0674x over previous
import functools

import jax
import jax.numpy as jnp
from jax import lax
from jax.experimental import pallas as pl
from jax.experimental.pallas import tpu as pltpu
from jax.experimental.pallas import tpu_sc as plsc

B = 16384
L = 20
D_USER, D_ITEM, D_HOUR, D_TEXT, D_PRICE = 20, 20, 5, 17, 10
D_OUT = 73
NC, NS, LANES = 2, 16, 16
NW = NC * NS
RPT = B // NW
HROW = 128
SENT = -1
DW = 32

_mesh = plsc.VectorSubcoreMesh(
    core_axis_name="c", subcore_axis_name="s", num_cores=NC, num_subcores=NS
)


def _iota16():
  return lax.iota(jnp.int32, LANES)


@functools.partial(
    pl.kernel,
    out_type=jax.ShapeDtypeStruct((B, D_OUT), jnp.float32),
    mesh=_mesh,
    compiler_params=pltpu.CompilerParams(
        needs_layout_passes=False, use_tc_tiling_on_sc=False,
        disable_bounds_checks=True),
    scratch_types=dict(
        idx_u=pltpu.VMEM((RPT,), jnp.int32),
        idx_i=pltpu.VMEM((RPT,), jnp.int32),
        idx_h=pltpu.VMEM((RPT,), jnp.int32),
        idx_p=pltpu.VMEM((RPT,), jnp.int32),
        it_a=pltpu.VMEM((RPT,), jnp.int32),
        it_b=pltpu.VMEM((RPT,), jnp.int32),
        is_a=pltpu.VMEM((RPT,), jnp.int32),
        is_b=pltpu.VMEM((RPT,), jnp.int32),
        tt=pltpu.VMEM((RPT * L,), jnp.int32),
        ctr_v=pltpu.VMEM((RPT,), jnp.float32),
        rows_u=pltpu.VMEM((RPT, DW), jnp.float32),
        rows_i=pltpu.VMEM((RPT, DW), jnp.float32),
        rows_p=pltpu.VMEM((RPT, 16), jnp.float32),
        tab_h=pltpu.VMEM((D_HOUR * (24 + 1),), jnp.float32),
        recip=pltpu.VMEM((RPT,), jnp.float32),
        out_v=pltpu.VMEM((RPT, D_OUT), jnp.float32),
        acc_s=pltpu.VMEM_SHARED((NS * RPT, DW), jnp.float32),
        sem_a=pltpu.SemaphoreType.DMA,
        sem_b=pltpu.SemaphoreType.DMA,
        sem_p=pltpu.SemaphoreType.DMA,
    ),
)
def _emb_kernel(W_user, W_item, W_hour, W_text, W_price, norm_ctr,
                cat_user_id, cat_item_id, int_hour, text_flat, disc_price,
                out_hbm, *, idx_u, idx_i, idx_h, idx_p,
                it_a, it_b, is_a, is_b, tt, ctr_v, rows_u, rows_i, rows_p,
                tab_h, recip, out_v, acc_s, sem_a, sem_b, sem_p):
  c = lax.axis_index("c")
  s = lax.axis_index("s")
  wid = c * NS + s
  base = wid * RPT
  sbase = s * RPT

  w_user, w_item, w_text, w_price = W_user, W_item, W_text, W_price

  pltpu.sync_copy(cat_user_id.at[pl.ds(base, RPT)], idx_u)
  pltpu.sync_copy(cat_item_id.at[pl.ds(base, RPT)], idx_i)
  pltpu.sync_copy(int_hour.at[pl.ds(base, RPT)], idx_h)
  pltpu.sync_copy(disc_price.at[pl.ds(base, RPT)], idx_p)
  pltpu.sync_copy(norm_ctr.at[pl.ds(base, RPT)], ctr_v)
  pltpu.sync_copy(text_flat.at[pl.ds(base * L, RPT * L)], tt)
  pltpu.sync_copy(W_hour, tab_h)

  iota = _iota16()

  def _zero_body(k, _):
    z = jnp.zeros((LANES,), jnp.float32)
    rows_u[k, pl.ds(0, LANES)] = z
    rows_u[k, pl.ds(DW - LANES, LANES)] = z
    return 0
  lax.fori_loop(0, RPT, _zero_body, 0)
  pltpu.sync_copy(rows_u, acc_s.at[pl.ds(sbase, RPT)])

  cp_u = pltpu.async_copy(w_user.at[idx_u], rows_u, sem_a)
  cp_i = pltpu.async_copy(w_item.at[idx_i], rows_i, sem_b)
  cp_p = pltpu.async_copy(w_price.at[idx_p], rows_p, sem_p)

  def _cnt_body(k, _):
    rr = (k * LANES + iota) * L
    cnt = jnp.zeros((LANES,), jnp.float32)
    for t in range(L):
      tok = plsc.load_gather(tt, [rr + t])
      cnt += jnp.where(tok != 0, 1.0, 0.0).astype(jnp.float32)
    recip[pl.ds(k * LANES, LANES)] = 1.0 / jnp.maximum(cnt, 1.0)
    return 0
  lax.fori_loop(0, RPT // LANES, _cnt_body, 0)

  def _copy_feat(src, d, col0):
    def body(i, _):
      lin = iota + i * LANES
      r = lax.div(lin, d)
      cc = lin - r * d
      v = plsc.load_gather(src, [r, cc])
      plsc.store_scatter(out_v, [r, col0 + cc], v)
      return 0
    lax.fori_loop(0, RPT * d // LANES, body, 0)

  cp_u.wait()
  _copy_feat(rows_u, D_USER, 0)
  cp_i.wait()
  _copy_feat(rows_i, D_ITEM, D_USER)
  cp_p.wait()
  _copy_feat(rows_p, D_PRICE, 62)

  def _hour_body(i, _):
    lin = iota + i * LANES
    r = lax.div(lin, D_HOUR)
    cc = lin - r * D_HOUR
    hh = plsc.load_gather(idx_h, [r])
    v = plsc.load_gather(tab_h, [hh * D_HOUR + cc])
    plsc.store_scatter(out_v, [r, 40 + cc], v)
    return 0
  lax.fori_loop(0, RPT * D_HOUR // LANES, _hour_body, 0)

  def _ctr_body(i, _):
    r = iota + i * LANES
    v = ctr_v[pl.ds(i * LANES, LANES)]
    plsc.store_scatter(out_v, [r, jnp.full((LANES,), D_OUT - 1, jnp.int32)], v)
    return 0
  lax.fori_loop(0, RPT // LANES, _ctr_body, 0)

  tbufs = (rows_u, rows_i)
  tsems = (sem_a, sem_b)
  tidx = (it_a, it_b)
  tsct = (is_a, is_b)

  def _gen_idx(t):
    def body(k, _):
      rloc = k * LANES + iota
      tok = plsc.load_gather(tt, [rloc * L + t])
      sl = pl.ds(k * LANES, LANES)
      tidx[t % 2][sl] = tok
      tsct[t % 2][sl] = jnp.where(tok == 0, jnp.int32(SENT), sbase + rloc)
      return 0
    lax.fori_loop(0, RPT // LANES, body, 0)

  _gen_idx(0)
  pend = [pltpu.async_copy(w_text.at[tidx[0]], tbufs[0], tsems[0])]
  _gen_idx(1)
  pend.append(pltpu.async_copy(w_text.at[tidx[1]], tbufs[1], tsems[1]))

  for t in range(L):
    pend.pop(0).wait()
    pltpu.sync_copy(
        tbufs[t % 2],
        acc_s.at[plsc.Indices(tsct[t % 2], ignored_value=SENT)],
        add=True)
    if t + 2 < L:
      _gen_idx(t + 2)
      pend.append(
          pltpu.async_copy(w_text.at[tidx[t % 2]], tbufs[t % 2], tsems[t % 2]))

  acc_v = rows_u
  pltpu.sync_copy(acc_s.at[pl.ds(sbase, RPT)], acc_v)

  def _text_body(i, _):
    lin = iota + i * LANES
    r = lax.div(lin, D_TEXT)
    cc = lin - r * D_TEXT
    a = plsc.load_gather(acc_v, [r, cc])
    rc = plsc.load_gather(recip, [r])
    plsc.store_scatter(out_v, [r, 45 + cc], a * rc)
    return 0
  lax.fori_loop(0, RPT * D_TEXT // LANES, _text_body, 0)

  pltpu.sync_copy(out_v, out_hbm.at[pl.ds(base, RPT)])


def kernel(W_user, W_item, W_hour, W_text, W_price, norm_ctr, cat_user_id,
           cat_item_id, int_hour, text_title, disc_clip_price):
  w_user = jnp.pad(W_user, ((0, 0), (0, DW - D_USER)))
  w_item = jnp.pad(W_item, ((0, 0), (0, DW - D_ITEM)))
  w_text = jnp.pad(W_text, ((0, 0), (0, DW - D_TEXT)))
  w_price = jnp.pad(W_price, ((0, 0), (0, 16 - D_PRICE)))
  return _emb_kernel(
      w_user, w_item, jnp.reshape(W_hour, (-1,)), w_text, w_price,
      norm_ctr, cat_user_id, cat_item_id, int_hour,
      jnp.reshape(text_title, (-1,)), disc_clip_price)
